# baseline (speedup 1.0000x reference)
import jax
import jax.numpy as jnp
from jax import lax
from jax.experimental import pallas as pl
from jax.experimental.pallas import tpu as pltpu
from jax.experimental.pallas import tpu_sc as plsc

_K = 26214
_N = 262144
_NS = 16
_SHARD = _N // _NS
_NVEC = _SHARD // 16
_NB = 2048
_NG = _NB // 16
_HR, _HC = 16, 128

_MININT = -2147483648


def _scan_hist(sh, scan_v, k):
    pltpu.sync_copy(sh, scan_v)
    iota = lax.iota(jnp.int32, 16)

    def body(i, carry):
        found, b, above, tot = carry
        v = _NG - 1 - i
        r = lax.shift_right_logical(v, 3)
        q = jnp.bitwise_and(v, 7)
        w = scan_v[r, pl.ds(q * 16, 16)]
        rw = lax.rev(w, (0,))
        cs = plsc.cumsum(rw)
        row_total = jnp.max(cs)
        anyf = (tot + row_total) >= k
        cond = (tot + cs) >= k
        j0 = plsc.all_reduce_ffs(cond)
        j0 = jnp.max(j0) if getattr(j0, "ndim", 0) else j0
        c_at = jnp.max(jnp.where(iota == j0, cs, 0))
        rw_at = jnp.max(jnp.where(iota == j0, rw, 0))
        hit = jnp.logical_and(jnp.logical_not(found), anyf)
        b = jnp.where(hit, v * 16 + 15 - j0, b)
        above = jnp.where(hit, tot + c_at - rw_at, above)
        return (jnp.logical_or(found, anyf), b, above, tot + row_total)

    _, b, above, _ = lax.fori_loop(
        0, _NG, body,
        (jnp.bool_(False), jnp.int32(0), jnp.int32(0), jnp.int32(0)))
    return b, above


def _sc_thr_body(mask_hbm, out_hbm, data_v, keys_v, hist_v, scan_v,
                 row_v, zero_v, outbuf_v, sh0, sh1, sh2):
    c = lax.axis_index("c")
    s = lax.axis_index("s")

    pltpu.sync_copy(mask_hbm.at[pl.ds(s * _SHARD, _SHARD)], data_v)

    iota = lax.iota(jnp.int32, 16)
    row_v[...] = iota
    for j in range(_HC // 16):
        zero_v[pl.ds(j * 16, 16)] = jnp.zeros((16,), jnp.int32)
    for sh in (sh0, sh1, sh2):
        pltpu.sync_copy(zero_v, sh.at[s])
    plsc.subcore_barrier()

    ones = jnp.ones((16,), jnp.int32)

    def merge(sh):
        plsc.subcore_barrier()
        pltpu.sync_copy(hist_v, sh.at[row_v], add=True)
        plsc.subcore_barrier()

    def zero_hist():
        def zh(j, _):
            def zc(q, _2):
                hist_v[j, pl.ds(q * 16, 16)] = jnp.zeros((16,), jnp.int32)
                return 0
            lax.fori_loop(0, _HC // 16, zc, 0)
            return 0
        lax.fori_loop(0, _HR, zh, 0)

    def scatter(b):
        plsc.addupdate_scatter(
            hist_v, [lax.shift_right_logical(b, 7), jnp.bitwise_and(b, 127)],
            ones)

    def scatter_masked(b, m):
        plsc.addupdate_scatter(
            hist_v, [lax.shift_right_logical(b, 7), jnp.bitwise_and(b, 127)],
            ones, mask=m)

    zero_hist()

    def p1(i, _):
        f = data_v[pl.ds(i * 16, 16)]
        bits = plsc.bitcast(f, jnp.int32)
        key = jnp.where(bits < 0, bits ^ jnp.int32(0x7FFFFFFF), bits)
        ukey = key ^ jnp.int32(_MININT)
        keys_v[pl.ds(i * 16, 16)] = ukey
        scatter(lax.shift_right_logical(ukey, 21))
        return 0

    lax.fori_loop(0, _NVEC, p1, 0)
    merge(sh0)
    p1v, above1 = _scan_hist(sh0, scan_v, jnp.int32(_K))
    k1 = _K - above1

    zero_hist()

    def p2(i, _):
        ukey = keys_v[pl.ds(i * 16, 16)]
        m = lax.shift_right_logical(ukey, 21) == p1v
        b2 = jnp.bitwise_and(lax.shift_right_logical(ukey, 10),
                             jnp.int32(0x7FF))
        scatter_masked(b2, m)
        return 0

    lax.fori_loop(0, _NVEC, p2, 0)
    merge(sh1)
    p2v, above2 = _scan_hist(sh1, scan_v, k1)
    k2 = k1 - above2

    pref22 = (p1v << 11) | p2v
    zero_hist()

    def p3(i, _):
        ukey = keys_v[pl.ds(i * 16, 16)]
        m = lax.shift_right_logical(ukey, 10) == pref22
        b3 = jnp.bitwise_and(ukey, jnp.int32(0x3FF))
        scatter_masked(b3, m)
        return 0

    lax.fori_loop(0, _NVEC, p3, 0)
    merge(sh2)
    p3v, _ = _scan_hist(sh2, scan_v, k2)

    ukey_t = (pref22 << 10) | p3v
    key_t = ukey_t ^ jnp.int32(_MININT)
    tbits = jnp.where(key_t >= 0, key_t, key_t ^ jnp.int32(0x7FFFFFFF))

    @pl.when(jnp.logical_and(c == 0, s == 0))
    def _write():
        outbuf_v[...] = jnp.broadcast_to(
            lax.bitcast_convert_type(tbits, jnp.float32), (16,))
        pltpu.sync_copy(outbuf_v, out_hbm)


def _sc_threshold(mask_flat):
    mesh = plsc.VectorSubcoreMesh(core_axis_name="c", subcore_axis_name="s")
    kfn = pl.kernel(
        _sc_thr_body,
        out_type=jax.ShapeDtypeStruct((16,), jnp.float32),
        mesh=mesh,
        compiler_params=pltpu.CompilerParams(needs_layout_passes=False),
        scratch_types=[
            pltpu.VMEM((_SHARD,), jnp.float32),
            pltpu.VMEM((_SHARD,), jnp.int32),
            pltpu.VMEM((_HR, _HC), jnp.int32),
            pltpu.VMEM((_HR, _HC), jnp.int32),
            pltpu.VMEM((16,), jnp.int32),
            pltpu.VMEM((_HC,), jnp.int32),
            pltpu.VMEM((16,), jnp.float32),
            pltpu.VMEM_SHARED((_HR, _HC), jnp.int32),
            pltpu.VMEM_SHARED((_HR, _HC), jnp.int32),
            pltpu.VMEM_SHARED((_HR, _HC), jnp.int32),
        ],
    )
    return kfn(mask_flat)


def _apply_kernel(thr_ref, mask_ref, x_ref, o_ref):
    t = (mask_ref[...] >= thr_ref[0]).astype(jnp.float32)
    o_ref[...] = x_ref[...] * (1.0 - 2.0 * t) + t


@jax.jit
def kernel(x, mask):
    b = x.shape[0]
    mflat = mask.reshape(_N)
    thr = _sc_threshold(mflat)

    out = pl.pallas_call(
        _apply_kernel,
        grid=(b,),
        in_specs=[
            pl.BlockSpec(memory_space=pltpu.SMEM),
            pl.BlockSpec((1, 64, 64, 64), lambda i: (0, 0, 0, 0)),
            pl.BlockSpec((1, 64, 64, 64), lambda i: (i, 0, 0, 0)),
        ],
        out_specs=pl.BlockSpec((1, 64, 64, 64), lambda i: (i, 0, 0, 0)),
        out_shape=jax.ShapeDtypeStruct(x.shape, jnp.float32),
    )(thr, mask, x)
    return out


# speedup vs baseline: 1.2196x; 1.2196x over previous
import jax
import jax.numpy as jnp
from jax import lax
from jax.experimental import pallas as pl
from jax.experimental.pallas import tpu as pltpu
from jax.experimental.pallas import tpu_sc as plsc

_K = 26214
_N = 262144
_NS = 16
_SHARD = _N // _NS
_NVEC = _SHARD // 16
_NB = 2048
_NG = _NB // 16
_HR, _HC = 16, 128

_MININT = -2147483648


def _scan_hist(sh, scan_v, k):
    pltpu.sync_copy(sh, scan_v)
    iota = lax.iota(jnp.int32, 16)

    rowtot = jnp.zeros((16,), jnp.int32)
    for r in range(_HR):
        acc = scan_v[r, pl.ds(0, 16)]
        for q in range(1, _HC // 16):
            acc = acc + scan_v[r, pl.ds(q * 16, 16)]
        rowtot = jnp.where(iota == r, jnp.sum(acc), rowtot)

    rrev = lax.rev(rowtot, (0,))
    rcs = plsc.cumsum(rrev)
    condr = rcs >= k
    j0r = plsc.all_reduce_ffs(condr)
    j0r = jnp.max(j0r) if getattr(j0r, "ndim", 0) else j0r
    rcs_at = jnp.max(jnp.where(iota == j0r, rcs, 0))
    rrev_at = jnp.max(jnp.where(iota == j0r, rrev, 0))
    r_star = 15 - j0r
    above_rows = rcs_at - rrev_at

    def body(i, carry):
        found, b, above, tot = carry
        q = (_HC // 16) - 1 - i
        w = scan_v[r_star, pl.ds(q * 16, 16)]
        rw = lax.rev(w, (0,))
        cs = plsc.cumsum(rw)
        chunk_total = jnp.max(cs)
        anyf = (tot + chunk_total) >= k
        cond = (tot + cs) >= k
        j0 = plsc.all_reduce_ffs(cond)
        j0 = jnp.max(j0) if getattr(j0, "ndim", 0) else j0
        c_at = jnp.max(jnp.where(iota == j0, cs, 0))
        rw_at = jnp.max(jnp.where(iota == j0, rw, 0))
        hit = jnp.logical_and(jnp.logical_not(found), anyf)
        b = jnp.where(hit, r_star * _HC + q * 16 + 15 - j0, b)
        above = jnp.where(hit, tot + c_at - rw_at, above)
        return (jnp.logical_or(found, anyf), b, above, tot + chunk_total)

    _, b, above, _ = lax.fori_loop(
        0, _HC // 16, body,
        (jnp.bool_(False), jnp.int32(0), jnp.int32(0), above_rows))
    return b, above


def _sc_thr_body(mask_hbm, out_hbm, data_v, keys_v, hist_v, scan_v,
                 row_v, zero_v, outbuf_v, sh0, sh1, sh2):
    c = lax.axis_index("c")
    s = lax.axis_index("s")

    pltpu.sync_copy(mask_hbm.at[pl.ds(s * _SHARD, _SHARD)], data_v)

    iota = lax.iota(jnp.int32, 16)
    row_v[...] = iota
    for j in range(_HC // 16):
        zero_v[pl.ds(j * 16, 16)] = jnp.zeros((16,), jnp.int32)
    for sh in (sh0, sh1, sh2):
        pltpu.sync_copy(zero_v, sh.at[s])
    plsc.subcore_barrier()

    ones = jnp.ones((16,), jnp.int32)

    def merge(sh):
        pltpu.sync_copy(hist_v, sh.at[row_v], add=True)
        plsc.subcore_barrier()

    def zero_hist():
        def zh(j, _):
            def zc(q, _2):
                hist_v[j, pl.ds(q * 16, 16)] = jnp.zeros((16,), jnp.int32)
                return 0
            lax.fori_loop(0, _HC // 16, zc, 0)
            return 0
        lax.fori_loop(0, _HR, zh, 0)

    def scatter(b):
        plsc.addupdate_scatter(
            hist_v, [lax.shift_right_logical(b, 7), jnp.bitwise_and(b, 127)],
            ones)

    def scatter_masked(b, m):
        plsc.addupdate_scatter(
            hist_v, [lax.shift_right_logical(b, 7), jnp.bitwise_and(b, 127)],
            ones, mask=m)

    zero_hist()

    @plsc.parallel_loop(0, _NVEC)
    def p1(i):
        f = data_v[pl.ds(i * 16, 16)]
        bits = plsc.bitcast(f, jnp.int32)
        key = jnp.where(bits < 0, bits ^ jnp.int32(0x7FFFFFFF), bits)
        ukey = key ^ jnp.int32(_MININT)
        keys_v[pl.ds(i * 16, 16)] = ukey
        scatter(lax.shift_right_logical(ukey, 21))

    merge(sh0)
    p1v, above1 = _scan_hist(sh0, scan_v, jnp.int32(_K))
    k1 = _K - above1

    zero_hist()

    @plsc.parallel_loop(0, _NVEC)
    def p2(i):
        ukey = keys_v[pl.ds(i * 16, 16)]
        m = lax.shift_right_logical(ukey, 21) == p1v
        b2 = jnp.bitwise_and(lax.shift_right_logical(ukey, 10),
                             jnp.int32(0x7FF))
        scatter_masked(b2, m)

    merge(sh1)
    p2v, above2 = _scan_hist(sh1, scan_v, k1)
    k2 = k1 - above2

    pref22 = (p1v << 11) | p2v
    zero_hist()

    @plsc.parallel_loop(0, _NVEC)
    def p3(i):
        ukey = keys_v[pl.ds(i * 16, 16)]
        m = lax.shift_right_logical(ukey, 10) == pref22
        b3 = jnp.bitwise_and(ukey, jnp.int32(0x3FF))
        scatter_masked(b3, m)

    merge(sh2)
    p3v, _ = _scan_hist(sh2, scan_v, k2)

    ukey_t = (pref22 << 10) | p3v
    key_t = ukey_t ^ jnp.int32(_MININT)
    tbits = jnp.where(key_t >= 0, key_t, key_t ^ jnp.int32(0x7FFFFFFF))

    @pl.when(jnp.logical_and(c == 0, s == 0))
    def _write():
        outbuf_v[...] = jnp.broadcast_to(
            lax.bitcast_convert_type(tbits, jnp.float32), (16,))
        pltpu.sync_copy(outbuf_v, out_hbm)


def _sc_threshold(mask_flat):
    mesh = plsc.VectorSubcoreMesh(core_axis_name="c", subcore_axis_name="s")
    kfn = pl.kernel(
        _sc_thr_body,
        out_type=jax.ShapeDtypeStruct((16,), jnp.float32),
        mesh=mesh,
        compiler_params=pltpu.CompilerParams(needs_layout_passes=False),
        scratch_types=[
            pltpu.VMEM((_SHARD,), jnp.float32),
            pltpu.VMEM((_SHARD,), jnp.int32),
            pltpu.VMEM((_HR, _HC), jnp.int32),
            pltpu.VMEM((_HR, _HC), jnp.int32),
            pltpu.VMEM((16,), jnp.int32),
            pltpu.VMEM((_HC,), jnp.int32),
            pltpu.VMEM((16,), jnp.float32),
            pltpu.VMEM_SHARED((_HR, _HC), jnp.int32),
            pltpu.VMEM_SHARED((_HR, _HC), jnp.int32),
            pltpu.VMEM_SHARED((_HR, _HC), jnp.int32),
        ],
    )
    return kfn(mask_flat)


def _apply_kernel(thr_ref, mask_ref, x_ref, o_ref):
    t = (mask_ref[...] >= thr_ref[0]).astype(jnp.float32)
    o_ref[...] = x_ref[...] * (1.0 - 2.0 * t) + t


@jax.jit
def kernel(x, mask):
    b = x.shape[0]
    mflat = mask.reshape(_N)
    thr = _sc_threshold(mflat)

    out = pl.pallas_call(
        _apply_kernel,
        grid=(b,),
        in_specs=[
            pl.BlockSpec(memory_space=pltpu.SMEM),
            pl.BlockSpec((1, 64, 64, 64), lambda i: (0, 0, 0, 0)),
            pl.BlockSpec((1, 64, 64, 64), lambda i: (i, 0, 0, 0)),
        ],
        out_specs=pl.BlockSpec((1, 64, 64, 64), lambda i: (i, 0, 0, 0)),
        out_shape=jax.ShapeDtypeStruct(x.shape, jnp.float32),
    )(thr, mask, x)
    return out


# speedup vs baseline: 1.3034x; 1.0687x over previous
import jax
import jax.numpy as jnp
from jax import lax
from jax.experimental import pallas as pl
from jax.experimental.pallas import tpu as pltpu
from jax.experimental.pallas import tpu_sc as plsc

_K = 26214
_N = 262144
_NS = 16
_SHARD = _N // _NS
_NVEC = _SHARD // 16
_NB = 2048
_NG = _NB // 16
_HR, _HC = 16, 128

_MININT = -2147483648


def _scan_hist(sh, scan_v, k):
    pltpu.sync_copy(sh, scan_v)
    iota = lax.iota(jnp.int32, 16)

    rowtot = jnp.zeros((16,), jnp.int32)
    for r in range(_HR):
        acc = scan_v[r, pl.ds(0, 16)]
        for q in range(1, _HC // 16):
            acc = acc + scan_v[r, pl.ds(q * 16, 16)]
        rowtot = jnp.where(iota == r, jnp.sum(acc), rowtot)

    rrev = lax.rev(rowtot, (0,))
    rcs = plsc.cumsum(rrev)
    condr = rcs >= k
    j0r = plsc.all_reduce_ffs(condr)
    j0r = jnp.max(j0r) if getattr(j0r, "ndim", 0) else j0r
    rcs_at = jnp.max(jnp.where(iota == j0r, rcs, 0))
    rrev_at = jnp.max(jnp.where(iota == j0r, rrev, 0))
    r_star = 15 - j0r
    above_rows = rcs_at - rrev_at

    def body(i, carry):
        found, b, above, tot = carry
        q = (_HC // 16) - 1 - i
        w = scan_v[r_star, pl.ds(q * 16, 16)]
        rw = lax.rev(w, (0,))
        cs = plsc.cumsum(rw)
        chunk_total = jnp.max(cs)
        anyf = (tot + chunk_total) >= k
        cond = (tot + cs) >= k
        j0 = plsc.all_reduce_ffs(cond)
        j0 = jnp.max(j0) if getattr(j0, "ndim", 0) else j0
        c_at = jnp.max(jnp.where(iota == j0, cs, 0))
        rw_at = jnp.max(jnp.where(iota == j0, rw, 0))
        hit = jnp.logical_and(jnp.logical_not(found), anyf)
        b = jnp.where(hit, r_star * _HC + q * 16 + 15 - j0, b)
        above = jnp.where(hit, tot + c_at - rw_at, above)
        return (jnp.logical_or(found, anyf), b, above, tot + chunk_total)

    _, b, above, _ = lax.fori_loop(
        0, _HC // 16, body,
        (jnp.bool_(False), jnp.int32(0), jnp.int32(0), above_rows))
    return b, above


def _sc_thr_body(mask_hbm, out_hbm, data_v, hist_v, scan_v,
                 row_v, zero_v, outbuf_v, sh0, sh1, sh2):
    c = lax.axis_index("c")
    s = lax.axis_index("s")

    pltpu.sync_copy(mask_hbm.at[pl.ds(s * _SHARD, _SHARD)], data_v)

    iota = lax.iota(jnp.int32, 16)
    row_v[...] = iota
    for j in range(_HC // 16):
        zero_v[pl.ds(j * 16, 16)] = jnp.zeros((16,), jnp.int32)
    for sh in (sh0, sh1, sh2):
        pltpu.sync_copy(zero_v, sh.at[s])
    plsc.subcore_barrier()

    ones = jnp.ones((16,), jnp.int32)

    def merge(sh):
        pltpu.sync_copy(hist_v, sh.at[row_v], add=True)
        plsc.subcore_barrier()

    def zero_hist():
        def zh(j, _):
            def zc(q, _2):
                hist_v[j, pl.ds(q * 16, 16)] = jnp.zeros((16,), jnp.int32)
                return 0
            lax.fori_loop(0, _HC // 16, zc, 0)
            return 0
        lax.fori_loop(0, _HR, zh, 0)

    def scatter(b):
        plsc.addupdate_scatter(
            hist_v, [lax.shift_right_logical(b, 7), jnp.bitwise_and(b, 127)],
            ones)

    def scatter_masked(b, m):
        plsc.addupdate_scatter(
            hist_v, [lax.shift_right_logical(b, 7), jnp.bitwise_and(b, 127)],
            ones, mask=m)

    zero_hist()

    def load_ukey(i):
        f = data_v[pl.ds(i * 16, 16)]
        bits = plsc.bitcast(f, jnp.int32)
        key = jnp.where(bits < 0, bits ^ jnp.int32(0x7FFFFFFF), bits)
        return key ^ jnp.int32(_MININT)

    @plsc.parallel_loop(0, _NVEC)
    def p1(i):
        scatter(lax.shift_right_logical(load_ukey(i), 21))

    merge(sh0)
    p1v, above1 = _scan_hist(sh0, scan_v, jnp.int32(_K))
    k1 = _K - above1

    zero_hist()

    @plsc.parallel_loop(0, _NVEC)
    def p2(i):
        ukey = load_ukey(i)
        m = lax.shift_right_logical(ukey, 21) == p1v
        b2 = jnp.bitwise_and(lax.shift_right_logical(ukey, 10),
                             jnp.int32(0x7FF))
        scatter_masked(b2, m)

    merge(sh1)
    p2v, above2 = _scan_hist(sh1, scan_v, k1)
    k2 = k1 - above2

    pref22 = (p1v << 11) | p2v
    zero_hist()

    @plsc.parallel_loop(0, _NVEC)
    def p3(i):
        ukey = load_ukey(i)
        m = lax.shift_right_logical(ukey, 10) == pref22
        b3 = jnp.bitwise_and(ukey, jnp.int32(0x3FF))
        scatter_masked(b3, m)

    merge(sh2)
    p3v, _ = _scan_hist(sh2, scan_v, k2)

    ukey_t = (pref22 << 10) | p3v
    key_t = ukey_t ^ jnp.int32(_MININT)
    tbits = jnp.where(key_t >= 0, key_t, key_t ^ jnp.int32(0x7FFFFFFF))

    @pl.when(jnp.logical_and(c == 0, s == 0))
    def _write():
        outbuf_v[...] = jnp.broadcast_to(
            lax.bitcast_convert_type(tbits, jnp.float32), (16,))
        pltpu.sync_copy(outbuf_v, out_hbm)


def _sc_threshold(mask_flat):
    mesh = plsc.VectorSubcoreMesh(core_axis_name="c", subcore_axis_name="s")
    kfn = pl.kernel(
        _sc_thr_body,
        out_type=jax.ShapeDtypeStruct((16,), jnp.float32),
        mesh=mesh,
        compiler_params=pltpu.CompilerParams(needs_layout_passes=False),
        scratch_types=[
            pltpu.VMEM((_SHARD,), jnp.float32),
            pltpu.VMEM((_HR, _HC), jnp.int32),
            pltpu.VMEM((_HR, _HC), jnp.int32),
            pltpu.VMEM((16,), jnp.int32),
            pltpu.VMEM((_HC,), jnp.int32),
            pltpu.VMEM((16,), jnp.float32),
            pltpu.VMEM_SHARED((_HR, _HC), jnp.int32),
            pltpu.VMEM_SHARED((_HR, _HC), jnp.int32),
            pltpu.VMEM_SHARED((_HR, _HC), jnp.int32),
        ],
    )
    return kfn(mask_flat)


def _apply_kernel(thr_ref, mask_ref, x_ref, o_ref):
    t = (mask_ref[...] >= thr_ref[0]).astype(jnp.float32)
    o_ref[...] = x_ref[...] * (1.0 - 2.0 * t) + t


@jax.jit
def kernel(x, mask):
    b = x.shape[0]
    mflat = mask.reshape(_N)
    thr = _sc_threshold(mflat)

    bb = 2
    out = pl.pallas_call(
        _apply_kernel,
        grid=(b // bb,),
        in_specs=[
            pl.BlockSpec(memory_space=pltpu.SMEM),
            pl.BlockSpec((1, 64, 64, 64), lambda i: (0, 0, 0, 0)),
            pl.BlockSpec((bb, 64, 64, 64), lambda i: (i, 0, 0, 0)),
        ],
        out_specs=pl.BlockSpec((bb, 64, 64, 64), lambda i: (i, 0, 0, 0)),
        out_shape=jax.ShapeDtypeStruct(x.shape, jnp.float32),
    )(thr, mask, x)
    return out


# speedup vs baseline: 1.3216x; 1.0139x over previous
import jax
import jax.numpy as jnp
from jax import lax
from jax.experimental import pallas as pl
from jax.experimental.pallas import tpu as pltpu
from jax.experimental.pallas import tpu_sc as plsc

_K = 26214
_N = 262144
_NS = 16
_SHARD = _N // _NS
_NVEC = _SHARD // 16
_NB = 2048
_NG = _NB // 16
_HR, _HC = 16, 128

_MININT = -2147483648


def _scan_hist(sh, scan_v, k):
    pltpu.sync_copy(sh, scan_v)
    iota = lax.iota(jnp.int32, 16)

    rowtot = jnp.zeros((16,), jnp.int32)
    for r in range(_HR):
        acc = scan_v[r, pl.ds(0, 16)]
        for q in range(1, _HC // 16):
            acc = acc + scan_v[r, pl.ds(q * 16, 16)]
        rowtot = jnp.where(iota == r, jnp.sum(acc), rowtot)

    rrev = lax.rev(rowtot, (0,))
    rcs = plsc.cumsum(rrev)
    condr = rcs >= k
    j0r = plsc.all_reduce_ffs(condr)
    j0r = jnp.max(j0r) if getattr(j0r, "ndim", 0) else j0r
    rcs_at = jnp.max(jnp.where(iota == j0r, rcs, 0))
    rrev_at = jnp.max(jnp.where(iota == j0r, rrev, 0))
    r_star = 15 - j0r
    above_rows = rcs_at - rrev_at

    def body(i, carry):
        found, b, above, tot = carry
        q = (_HC // 16) - 1 - i
        w = scan_v[r_star, pl.ds(q * 16, 16)]
        rw = lax.rev(w, (0,))
        cs = plsc.cumsum(rw)
        chunk_total = jnp.max(cs)
        anyf = (tot + chunk_total) >= k
        cond = (tot + cs) >= k
        j0 = plsc.all_reduce_ffs(cond)
        j0 = jnp.max(j0) if getattr(j0, "ndim", 0) else j0
        c_at = jnp.max(jnp.where(iota == j0, cs, 0))
        rw_at = jnp.max(jnp.where(iota == j0, rw, 0))
        hit = jnp.logical_and(jnp.logical_not(found), anyf)
        b = jnp.where(hit, r_star * _HC + q * 16 + 15 - j0, b)
        above = jnp.where(hit, tot + c_at - rw_at, above)
        return (jnp.logical_or(found, anyf), b, above, tot + chunk_total)

    _, b, above, _ = lax.fori_loop(
        0, _HC // 16, body,
        (jnp.bool_(False), jnp.int32(0), jnp.int32(0), above_rows))
    return b, above


def _sc_thr_body(mask_hbm, out_hbm, data_v, hist_v, scan_v,
                 row_v, zero_v, outbuf_v, sh0, sh1, sh2):
    c = lax.axis_index("c")
    s = lax.axis_index("s")

    pltpu.sync_copy(mask_hbm.at[pl.ds(s * _SHARD, _SHARD)], data_v)

    iota = lax.iota(jnp.int32, 16)
    row_v[...] = iota
    for j in range(_HC // 16):
        zero_v[pl.ds(j * 16, 16)] = jnp.zeros((16,), jnp.int32)
    for sh in (sh0, sh1, sh2):
        pltpu.sync_copy(zero_v, sh.at[s])
    plsc.subcore_barrier()

    ones = jnp.ones((16,), jnp.int32)

    def merge(sh):
        pltpu.sync_copy(hist_v, sh.at[row_v], add=True)
        plsc.subcore_barrier()

    def zero_hist():
        def zh(j, _):
            def zc(q, _2):
                hist_v[j, pl.ds(q * 16, 16)] = jnp.zeros((16,), jnp.int32)
                return 0
            lax.fori_loop(0, _HC // 16, zc, 0)
            return 0
        lax.fori_loop(0, _HR, zh, 0)

    def scatter(b):
        plsc.addupdate_scatter(
            hist_v, [lax.shift_right_logical(b, 7), jnp.bitwise_and(b, 127)],
            ones)

    def scatter_masked(b, m):
        plsc.addupdate_scatter(
            hist_v, [lax.shift_right_logical(b, 7), jnp.bitwise_and(b, 127)],
            ones, mask=m)

    zero_hist()

    def load_ukey(i):
        f = data_v[pl.ds(i * 16, 16)]
        bits = plsc.bitcast(f, jnp.int32)
        key = jnp.where(bits < 0, bits ^ jnp.int32(0x7FFFFFFF), bits)
        return key ^ jnp.int32(_MININT)

    @plsc.parallel_loop(0, _NVEC)
    def p1(i):
        scatter(lax.shift_right_logical(load_ukey(i), 21))

    merge(sh0)
    p1v, above1 = _scan_hist(sh0, scan_v, jnp.int32(_K))
    k1 = _K - above1

    zero_hist()

    @plsc.parallel_loop(0, _NVEC)
    def p2(i):
        ukey = load_ukey(i)
        m = lax.shift_right_logical(ukey, 21) == p1v
        b2 = jnp.bitwise_and(lax.shift_right_logical(ukey, 10),
                             jnp.int32(0x7FF))
        scatter_masked(b2, m)

    merge(sh1)
    p2v, above2 = _scan_hist(sh1, scan_v, k1)
    k2 = k1 - above2

    pref22 = (p1v << 11) | p2v
    zero_hist()

    @plsc.parallel_loop(0, _NVEC)
    def p3(i):
        ukey = load_ukey(i)
        m = lax.shift_right_logical(ukey, 10) == pref22
        b3 = jnp.bitwise_and(ukey, jnp.int32(0x3FF))
        scatter_masked(b3, m)

    merge(sh2)
    p3v, _ = _scan_hist(sh2, scan_v, k2)

    ukey_t = (pref22 << 10) | p3v
    key_t = ukey_t ^ jnp.int32(_MININT)
    tbits = jnp.where(key_t >= 0, key_t, key_t ^ jnp.int32(0x7FFFFFFF))

    @pl.when(jnp.logical_and(c == 0, s == 0))
    def _write():
        outbuf_v[...] = jnp.broadcast_to(
            lax.bitcast_convert_type(tbits, jnp.float32), (16,))
        pltpu.sync_copy(outbuf_v, out_hbm)


def _sc_threshold(mask_flat):
    mesh = plsc.VectorSubcoreMesh(core_axis_name="c", subcore_axis_name="s")
    kfn = pl.kernel(
        _sc_thr_body,
        out_type=jax.ShapeDtypeStruct((16,), jnp.float32),
        mesh=mesh,
        compiler_params=pltpu.CompilerParams(needs_layout_passes=False),
        scratch_types=[
            pltpu.VMEM((_SHARD,), jnp.float32),
            pltpu.VMEM((_HR, _HC), jnp.int32),
            pltpu.VMEM((_HR, _HC), jnp.int32),
            pltpu.VMEM((16,), jnp.int32),
            pltpu.VMEM((_HC,), jnp.int32),
            pltpu.VMEM((16,), jnp.float32),
            pltpu.VMEM_SHARED((_HR, _HC), jnp.int32),
            pltpu.VMEM_SHARED((_HR, _HC), jnp.int32),
            pltpu.VMEM_SHARED((_HR, _HC), jnp.int32),
        ],
    )
    return kfn(mask_flat)


def _apply_kernel(thr_ref, mask_ref, x_ref, o_ref):
    t = (mask_ref[...] >= thr_ref[0]).astype(jnp.float32)
    o_ref[...] = x_ref[...] * (1.0 - 2.0 * t) + t


@jax.jit
def kernel(x, mask):
    b = x.shape[0]
    mflat = mask.reshape(_N)
    thr = _sc_threshold(mflat)

    bb = 4
    out = pl.pallas_call(
        _apply_kernel,
        grid=(b // bb,),
        in_specs=[
            pl.BlockSpec(memory_space=pltpu.SMEM),
            pl.BlockSpec((1, 64, 64, 64), lambda i: (0, 0, 0, 0)),
            pl.BlockSpec((bb, 64, 64, 64), lambda i: (i, 0, 0, 0)),
        ],
        out_specs=pl.BlockSpec((bb, 64, 64, 64), lambda i: (i, 0, 0, 0)),
        out_shape=jax.ShapeDtypeStruct(x.shape, jnp.float32),
    )(thr, mask, x)
    return out
